# baseline (speedup 1.0000x reference)
import functools

import jax
import jax.numpy as jnp
from jax import lax
from jax.experimental import pallas as pl
from jax.experimental.pallas import tpu as pltpu
from jax.experimental.pallas import tpu_sc as plsc

N_NODES = 10000
N_EDGES = 320000
D = 128

NC = 2
NS = 16
NW = NC * NS
CH = 64
NBUF = 4
DGSZ = 16
CPW = 160
NGRP = CPW // NBUF
E_PAD = NW * CH * CPW
EPW = E_PAD // NW
NCHT = E_PAD // CH
NP = 10240
ZROWS = NP // NS
BLK = 512
GRID = NP // BLK

_f32 = jnp.float32


def _make_deg_kernel():
    mesh = plsc.VectorSubcoreMesh(core_axis_name="c", subcore_axis_name="s")

    @functools.partial(
        pl.kernel,
        out_type=jax.ShapeDtypeStruct((NC, NP), _f32),
        mesh=mesh,
        scratch_types=[
            pltpu.VMEM((DGSZ, CH), jnp.int32),
            pltpu.VMEM((DGSZ, CH), jnp.int32),
            pltpu.VMEM((CH,), _f32),
            pltpu.VMEM((ZROWS,), _f32),
            pltpu.VMEM_SHARED((NP,), _f32),
            pltpu.SemaphoreType.DMA,
            pltpu.SemaphoreType.DMA,
            pltpu.SemaphoreType.DMA,
            pltpu.SemaphoreType.DMA,
            pltpu.SemaphoreType.DMA,
            pltpu.SemaphoreType.DMA,
        ],
    )
    def deg_kernel(dst_hbm, out_hbm, dstA, dstB, ones_v, zeros_v, acc_sh,
                   sia, sib, s0, s1, s2, s3):
        c = lax.axis_index("c")
        s = lax.axis_index("s")
        wid = s * NC + c
        ssem = (s0, s1, s2, s3)

        one = jnp.full((16,), 1.0, _f32)
        zero = jnp.zeros((16,), _f32)

        def fill_ones(i, _):
            ones_v[pl.ds(i * 16, 16)] = one
            return 0

        lax.fori_loop(0, CH // 16, fill_ones, 0)

        def fill_zeros(i, _):
            zeros_v[pl.ds(i * 16, 16)] = zero
            return 0

        lax.fori_loop(0, ZROWS // 16, fill_zeros, 0)
        pltpu.sync_copy(zeros_v, acc_sh.at[pl.ds(s * ZROWS, ZROWS)])
        plsc.subcore_barrier()

        base = wid * CPW
        pltpu.async_copy(dst_hbm.at[pl.ds(base, DGSZ)], dstA, sia).wait()

        def body(k, _):
            dib = pltpu.async_copy(
                dst_hbm.at[pl.ds(base + (2 * k + 1) * DGSZ, DGSZ)], dstB, sib)
            dsa = [pltpu.async_copy(ones_v, acc_sh.at[dstA.at[j]],
                                    ssem[j % 4], add=True)
                   for j in range(DGSZ)]
            dib.wait()
            for j in range(DGSZ):
                dsa[j].wait()
            nxt = jnp.minimum(base + (2 * k + 2) * DGSZ, NCHT - DGSZ)
            dia = pltpu.async_copy(dst_hbm.at[pl.ds(nxt, DGSZ)], dstA, sia)
            dsb = [pltpu.async_copy(ones_v, acc_sh.at[dstB.at[j]],
                                    ssem[j % 4], add=True)
                   for j in range(DGSZ)]
            dia.wait()
            for j in range(DGSZ):
                dsb[j].wait()
            return 0

        lax.fori_loop(0, CPW // DGSZ // 2, body, 0)
        plsc.subcore_barrier()
        pltpu.sync_copy(acc_sh.at[pl.ds(s * ZROWS, ZROWS)],
                        out_hbm.at[c, pl.ds(s * ZROWS, ZROWS)])

    return deg_kernel


def _make_agg_kernel():
    mesh = plsc.VectorSubcoreMesh(core_axis_name="c", subcore_axis_name="s")

    @functools.partial(
        pl.kernel,
        out_type=jax.ShapeDtypeStruct((NC, NP, D), _f32),
        mesh=mesh,
        scratch_types=[
            pltpu.VMEM((NBUF, CH), jnp.int32),
            pltpu.VMEM((NBUF, CH), jnp.int32),
            pltpu.VMEM((NBUF, CH), jnp.int32),
            pltpu.VMEM((NBUF, CH), jnp.int32),
            pltpu.VMEM((NBUF, CH, D), _f32),
            pltpu.VMEM_SHARED((NP, D), _f32),
            pltpu.SemaphoreType.DMA,
            pltpu.SemaphoreType.DMA,
            pltpu.SemaphoreType.DMA,
            pltpu.SemaphoreType.DMA,
            pltpu.SemaphoreType.DMA,
            pltpu.SemaphoreType.DMA,
            pltpu.SemaphoreType.DMA,
            pltpu.SemaphoreType.DMA,
            pltpu.SemaphoreType.DMA,
            pltpu.SemaphoreType.DMA,
        ],
    )
    def agg_kernel(y_hbm, src_hbm, dst_hbm, out_hbm,
                   srcA, srcB, dstA, dstB, rows_v, acc_sh,
                   sia, sib, g0, g1, g2, g3, s0, s1, s2, s3):
        c = lax.axis_index("c")
        s = lax.axis_index("s")
        wid = s * NC + c
        gsem = (g0, g1, g2, g3)
        ssem = (s0, s1, s2, s3)

        zero = jnp.zeros((16,), _f32)

        def fill_zeros(i, _):
            rows_v[0, i // (D // 16), pl.ds((i % (D // 16)) * 16, 16)] = zero
            return 0

        lax.fori_loop(0, CH * (D // 16), fill_zeros, 0)

        def zero_acc(i, _):
            pltpu.async_copy(rows_v.at[0],
                             acc_sh.at[pl.ds(s * ZROWS + i * CH, CH)],
                             gsem[0]).wait()
            return 0

        def init_from_y(i, _):
            r = s * ZROWS + i * CH
            pltpu.async_copy(y_hbm.at[pl.ds(r, CH)], acc_sh.at[pl.ds(r, CH)],
                             gsem[0]).wait()
            return 0

        @pl.when(c == 0)
        def _():
            lax.fori_loop(0, ZROWS // CH, init_from_y, 0)

        @pl.when(c != 0)
        def _():
            lax.fori_loop(0, ZROWS // CH, zero_acc, 0)

        lanes = lax.iota(jnp.int32, 16)
        for t in range(CH // 16):
            dstB[0, pl.ds(t * 16, 16)] = lanes + (N_NODES + (t * 16) % 128)
        plsc.subcore_barrier()

        prime = [pltpu.async_copy(rows_v.at[j], acc_sh.at[dstB.at[0]],
                                  ssem[j], add=True) for j in range(NBUF)]
        del prime

        def wait_scatter(j):
            pltpu.make_async_copy(rows_v.at[j], acc_sh.at[dstB.at[0]],
                                  ssem[j]).wait()

        base = wid * CPW
        da = pltpu.async_copy(src_hbm.at[pl.ds(base, NBUF)], srcA, sia)
        db = pltpu.async_copy(dst_hbm.at[pl.ds(base, NBUF)], dstA, sia)
        da.wait()
        db.wait()

        def body(k, _):
            dg = []
            for j in range(NBUF):
                wait_scatter(j)
                dg.append(pltpu.async_copy(y_hbm.at[srcA.at[j]],
                                           rows_v.at[j], gsem[j]))
            cb = base + (2 * k + 1) * NBUF
            dib1 = pltpu.async_copy(src_hbm.at[pl.ds(cb, NBUF)], srcB, sib)
            dib2 = pltpu.async_copy(dst_hbm.at[pl.ds(cb, NBUF)], dstB, sib)
            for j in range(NBUF):
                dg[j].wait()
                pltpu.async_copy(rows_v.at[j], acc_sh.at[dstA.at[j]],
                                 ssem[j], add=True)
            dib1.wait()
            dib2.wait()
            dg = []
            for j in range(NBUF):
                wait_scatter(j)
                dg.append(pltpu.async_copy(y_hbm.at[srcB.at[j]],
                                           rows_v.at[j], gsem[j]))
            nxt = jnp.minimum(base + (2 * k + 2) * NBUF, NCHT - NBUF)
            dia1 = pltpu.async_copy(src_hbm.at[pl.ds(nxt, NBUF)], srcA, sia)
            dia2 = pltpu.async_copy(dst_hbm.at[pl.ds(nxt, NBUF)], dstA, sia)
            for j in range(NBUF):
                dg[j].wait()
                pltpu.async_copy(rows_v.at[j], acc_sh.at[dstB.at[j]],
                                 ssem[j], add=True)
            dia1.wait()
            dia2.wait()
            return 0

        lax.fori_loop(0, NGRP // 2, body, 0)
        for j in range(NBUF):
            wait_scatter(j)
        plsc.subcore_barrier()
        pltpu.sync_copy(acc_sh.at[pl.ds(s * ZROWS, ZROWS)],
                        out_hbm.at[c, pl.ds(s * ZROWS, ZROWS)])

    return agg_kernel


def _dinv_block(deg_ref):
    deg = deg_ref[0, :] + deg_ref[1, :] + 1.0
    return lax.rsqrt(deg)


def _mm1_body(deg_ref, x_ref, w_ref, y_ref):
    dinv = _dinv_block(deg_ref)
    xw = jnp.dot(x_ref[...], w_ref[...], preferred_element_type=_f32)
    y_ref[...] = xw * dinv[:, None]


def _mm2_body(deg_ref, agg_ref, w_ref, b_ref, y2_ref):
    dinv = _dinv_block(deg_ref)
    pre = (agg_ref[0] + agg_ref[1]) * dinv[:, None] + b_ref[...]
    h = jnp.maximum(pre, 0.0)
    hw = jnp.dot(h, w_ref[...], preferred_element_type=_f32)
    y2_ref[...] = hw * dinv[:, None]


def _mm3_body(deg_ref, agg_ref, b_ref, z_ref):
    dinv = _dinv_block(deg_ref)
    z_ref[...] = (agg_ref[0] + agg_ref[1]) * dinv[:, None] + b_ref[...]


_deg_spec = pl.BlockSpec((NC, BLK), lambda i: (0, i))
_row_spec = pl.BlockSpec((BLK, D), lambda i: (i, 0))
_w_spec = pl.BlockSpec((D, D), lambda i: (0, 0))
_b_spec = pl.BlockSpec((1, D), lambda i: (0, 0))
_agg_spec = pl.BlockSpec((NC, BLK, D), lambda i: (0, i, 0))
_out_shape = jax.ShapeDtypeStruct((NP, D), _f32)


def _mm1(degp, x, w):
    return pl.pallas_call(
        _mm1_body, grid=(GRID,),
        in_specs=[_deg_spec, _row_spec, _w_spec],
        out_specs=_row_spec, out_shape=_out_shape,
    )(degp, x, w)


def _mm2(degp, agg, w, b):
    return pl.pallas_call(
        _mm2_body, grid=(GRID,),
        in_specs=[_deg_spec, _agg_spec, _w_spec, _b_spec],
        out_specs=_row_spec, out_shape=_out_shape,
    )(degp, agg, w, b)


def _mm3(degp, agg, b):
    return pl.pallas_call(
        _mm3_body, grid=(GRID,),
        in_specs=[_deg_spec, _agg_spec, _b_spec],
        out_specs=_row_spec, out_shape=_out_shape,
    )(degp, agg, b)


def kernel(x, edge_index, W1, b1, W2, b2):
    src = edge_index[0].astype(jnp.int32)
    dst = edge_index[1].astype(jnp.int32)
    pad = E_PAD - N_EDGES
    it = jnp.arange(pad, dtype=jnp.int32)
    src_p = jnp.concatenate([src, it % N_NODES]).reshape(NCHT, CH)
    dst_p = jnp.concatenate([dst, N_NODES + it % (NP - N_NODES)]
                            ).reshape(NCHT, CH)
    b1r = b1.reshape(1, D)
    b2r = b2.reshape(1, D)

    deg_k = _make_deg_kernel()
    agg_k = _make_agg_kernel()

    x_p = jnp.pad(x, ((0, NP - N_NODES), (0, 0)))
    degp = deg_k(dst_p)
    y1 = _mm1(degp, x_p, W1)
    agg1 = agg_k(y1, src_p, dst_p)
    y2 = _mm2(degp, agg1, W2, b1r)
    agg2 = agg_k(y2, src_p, dst_p)
    return _mm3(degp, agg2, b2r)[:N_NODES]


# speedup vs baseline: 1.0381x; 1.0381x over previous
import functools

import jax
import jax.numpy as jnp
from jax import lax
from jax.experimental import pallas as pl
from jax.experimental.pallas import tpu as pltpu
from jax.experimental.pallas import tpu_sc as plsc

N_NODES = 10000
N_EDGES = 320000
D = 128

NC = 2
NS = 16
NW = NC * NS
CH = 64
NBUF = 4
DGSZ = 16
CPW = 160
NGRP = CPW // NBUF
E_PAD = NW * CH * CPW
EPW = E_PAD // NW
NCHT = E_PAD // CH
NP = 10240
ZROWS = NP // NS
BLK = 512
GRID = NP // BLK

_f32 = jnp.float32


def _make_deg_kernel():
    mesh = plsc.VectorSubcoreMesh(core_axis_name="c", subcore_axis_name="s")

    @functools.partial(
        pl.kernel,
        out_type=jax.ShapeDtypeStruct((NC, NP), _f32),
        mesh=mesh,
        scratch_types=[
            pltpu.VMEM((DGSZ, CH), jnp.int32),
            pltpu.VMEM((DGSZ, CH), jnp.int32),
            pltpu.VMEM((CH,), _f32),
            pltpu.VMEM((ZROWS,), _f32),
            pltpu.VMEM_SHARED((NP,), _f32),
            pltpu.SemaphoreType.DMA,
            pltpu.SemaphoreType.DMA,
            pltpu.SemaphoreType.DMA,
            pltpu.SemaphoreType.DMA,
            pltpu.SemaphoreType.DMA,
            pltpu.SemaphoreType.DMA,
        ],
    )
    def deg_kernel(dst_hbm, out_hbm, dstA, dstB, ones_v, zeros_v, acc_sh,
                   sia, sib, s0, s1, s2, s3):
        c = lax.axis_index("c")
        s = lax.axis_index("s")
        wid = s * NC + c
        ssem = (s0, s1, s2, s3)

        one = jnp.full((16,), 1.0, _f32)
        zero = jnp.zeros((16,), _f32)

        def fill_ones(i, _):
            ones_v[pl.ds(i * 16, 16)] = one
            return 0

        lax.fori_loop(0, CH // 16, fill_ones, 0)

        def fill_zeros(i, _):
            zeros_v[pl.ds(i * 16, 16)] = zero
            return 0

        lax.fori_loop(0, ZROWS // 16, fill_zeros, 0)
        pltpu.sync_copy(zeros_v, acc_sh.at[pl.ds(s * ZROWS, ZROWS)])
        plsc.subcore_barrier()

        base = wid * CPW
        pltpu.async_copy(dst_hbm.at[pl.ds(base, DGSZ)], dstA, sia).wait()

        def body(k, _):
            dib = pltpu.async_copy(
                dst_hbm.at[pl.ds(base + (2 * k + 1) * DGSZ, DGSZ)], dstB, sib)
            dsa = [pltpu.async_copy(ones_v, acc_sh.at[dstA.at[j]],
                                    ssem[j % 4], add=True)
                   for j in range(DGSZ)]
            dib.wait()
            for j in range(DGSZ):
                dsa[j].wait()
            nxt = jnp.minimum(base + (2 * k + 2) * DGSZ, NCHT - DGSZ)
            dia = pltpu.async_copy(dst_hbm.at[pl.ds(nxt, DGSZ)], dstA, sia)
            dsb = [pltpu.async_copy(ones_v, acc_sh.at[dstB.at[j]],
                                    ssem[j % 4], add=True)
                   for j in range(DGSZ)]
            dia.wait()
            for j in range(DGSZ):
                dsb[j].wait()
            return 0

        lax.fori_loop(0, CPW // DGSZ // 2, body, 0)
        plsc.subcore_barrier()
        pltpu.sync_copy(acc_sh.at[pl.ds(s * ZROWS, ZROWS)],
                        out_hbm.at[c, pl.ds(s * ZROWS, ZROWS)])

    return deg_kernel


def _make_agg_kernel():
    mesh = plsc.VectorSubcoreMesh(core_axis_name="c", subcore_axis_name="s")

    @functools.partial(
        pl.kernel,
        out_type=jax.ShapeDtypeStruct((NC, NP, D), _f32),
        mesh=mesh,
        scratch_types=[
            pltpu.VMEM((NBUF, CH), jnp.int32),
            pltpu.VMEM((NBUF, CH), jnp.int32),
            pltpu.VMEM((NBUF, CH), jnp.int32),
            pltpu.VMEM((NBUF, CH), jnp.int32),
            pltpu.VMEM((NBUF, CH, D), _f32),
            pltpu.VMEM_SHARED((NP, D), _f32),
            pltpu.SemaphoreType.DMA,
            pltpu.SemaphoreType.DMA,
            pltpu.SemaphoreType.DMA,
            pltpu.SemaphoreType.DMA,
            pltpu.SemaphoreType.DMA,
            pltpu.SemaphoreType.DMA,
            pltpu.SemaphoreType.DMA,
            pltpu.SemaphoreType.DMA,
            pltpu.SemaphoreType.DMA,
            pltpu.SemaphoreType.DMA,
        ],
    )
    def agg_kernel(y_hbm, src_hbm, dst_hbm, out_hbm,
                   srcA, srcB, dstA, dstB, rows_v, acc_sh,
                   sia, sib, g0, g1, g2, g3, s0, s1, s2, s3):
        c = lax.axis_index("c")
        s = lax.axis_index("s")
        wid = s * NC + c
        gsem = (g0, g1, g2, g3)
        ssem = (s0, s1, s2, s3)

        zero = jnp.zeros((16,), _f32)

        def fill_zeros(i, _):
            rows_v[0, i // (D // 16), pl.ds((i % (D // 16)) * 16, 16)] = zero
            return 0

        lax.fori_loop(0, CH * (D // 16), fill_zeros, 0)

        zd = [pltpu.async_copy(rows_v.at[0],
                               acc_sh.at[pl.ds(s * ZROWS + i * CH, CH)],
                               gsem[i % NBUF])
              for i in range(ZROWS // CH)]
        for d in zd:
            d.wait()

        lanes = lax.iota(jnp.int32, 16)
        for t in range(CH // 16):
            dstB[0, pl.ds(t * 16, 16)] = lanes + (N_NODES + (t * 16) % 128)
        plsc.subcore_barrier()

        prime = [pltpu.async_copy(rows_v.at[j], acc_sh.at[dstB.at[0]],
                                  ssem[j], add=True) for j in range(NBUF)]
        del prime

        def wait_scatter(j):
            pltpu.make_async_copy(rows_v.at[j], acc_sh.at[dstB.at[0]],
                                  ssem[j]).wait()

        base = wid * CPW
        da = pltpu.async_copy(src_hbm.at[pl.ds(base, NBUF)], srcA, sia)
        db = pltpu.async_copy(dst_hbm.at[pl.ds(base, NBUF)], dstA, sia)
        da.wait()
        db.wait()

        def body(k, _):
            dg = []
            for j in range(NBUF):
                wait_scatter(j)
                dg.append(pltpu.async_copy(y_hbm.at[srcA.at[j]],
                                           rows_v.at[j], gsem[j]))
            cb = base + (2 * k + 1) * NBUF
            dib1 = pltpu.async_copy(src_hbm.at[pl.ds(cb, NBUF)], srcB, sib)
            dib2 = pltpu.async_copy(dst_hbm.at[pl.ds(cb, NBUF)], dstB, sib)
            for j in range(NBUF):
                dg[j].wait()
                pltpu.async_copy(rows_v.at[j], acc_sh.at[dstA.at[j]],
                                 ssem[j], add=True)
            dib1.wait()
            dib2.wait()
            dg = []
            for j in range(NBUF):
                wait_scatter(j)
                dg.append(pltpu.async_copy(y_hbm.at[srcB.at[j]],
                                           rows_v.at[j], gsem[j]))
            nxt = jnp.minimum(base + (2 * k + 2) * NBUF, NCHT - NBUF)
            dia1 = pltpu.async_copy(src_hbm.at[pl.ds(nxt, NBUF)], srcA, sia)
            dia2 = pltpu.async_copy(dst_hbm.at[pl.ds(nxt, NBUF)], dstA, sia)
            for j in range(NBUF):
                dg[j].wait()
                pltpu.async_copy(rows_v.at[j], acc_sh.at[dstB.at[j]],
                                 ssem[j], add=True)
            dia1.wait()
            dia2.wait()
            return 0

        lax.fori_loop(0, NGRP // 2, body, 0)
        for j in range(NBUF):
            wait_scatter(j)
        plsc.subcore_barrier()
        pltpu.sync_copy(acc_sh.at[pl.ds(s * ZROWS, ZROWS)],
                        out_hbm.at[c, pl.ds(s * ZROWS, ZROWS)])

    return agg_kernel


def _dinv_block(deg_ref):
    deg = deg_ref[0, :] + deg_ref[1, :] + 1.0
    return lax.rsqrt(deg)


def _mm1_body(deg_ref, x_ref, w_ref, y_ref):
    dinv = _dinv_block(deg_ref)
    xw = jnp.dot(x_ref[...], w_ref[...], preferred_element_type=_f32)
    y_ref[...] = xw * dinv[:, None]


def _mm2_body(deg_ref, agg_ref, y1_ref, w_ref, b_ref, y2_ref):
    dinv = _dinv_block(deg_ref)
    pre = (agg_ref[0] + agg_ref[1] + y1_ref[...]) * dinv[:, None] + b_ref[...]
    h = jnp.maximum(pre, 0.0)
    hw = jnp.dot(h, w_ref[...], preferred_element_type=_f32)
    y2_ref[...] = hw * dinv[:, None]


def _mm3_body(deg_ref, agg_ref, y2_ref, b_ref, z_ref):
    dinv = _dinv_block(deg_ref)
    z_ref[...] = ((agg_ref[0] + agg_ref[1] + y2_ref[...]) * dinv[:, None]
                  + b_ref[...])


_deg_spec = pl.BlockSpec((NC, BLK), lambda i: (0, i))
_row_spec = pl.BlockSpec((BLK, D), lambda i: (i, 0))
_w_spec = pl.BlockSpec((D, D), lambda i: (0, 0))
_b_spec = pl.BlockSpec((1, D), lambda i: (0, 0))
_agg_spec = pl.BlockSpec((NC, BLK, D), lambda i: (0, i, 0))
_out_shape = jax.ShapeDtypeStruct((NP, D), _f32)


def _mm1(degp, x, w):
    return pl.pallas_call(
        _mm1_body, grid=(GRID,),
        in_specs=[_deg_spec, _row_spec, _w_spec],
        out_specs=_row_spec, out_shape=_out_shape,
    )(degp, x, w)


def _mm2(degp, agg, y1, w, b):
    return pl.pallas_call(
        _mm2_body, grid=(GRID,),
        in_specs=[_deg_spec, _agg_spec, _row_spec, _w_spec, _b_spec],
        out_specs=_row_spec, out_shape=_out_shape,
    )(degp, agg, y1, w, b)


def _mm3(degp, agg, y2, b):
    return pl.pallas_call(
        _mm3_body, grid=(GRID,),
        in_specs=[_deg_spec, _agg_spec, _row_spec, _b_spec],
        out_specs=_row_spec, out_shape=_out_shape,
    )(degp, agg, y2, b)


def kernel(x, edge_index, W1, b1, W2, b2):
    src = edge_index[0].astype(jnp.int32)
    dst = edge_index[1].astype(jnp.int32)
    pad = E_PAD - N_EDGES
    it = jnp.arange(pad, dtype=jnp.int32)
    src_p = jnp.concatenate([src, it % N_NODES]).reshape(NCHT, CH)
    dst_p = jnp.concatenate([dst, N_NODES + it % (NP - N_NODES)]
                            ).reshape(NCHT, CH)
    b1r = b1.reshape(1, D)
    b2r = b2.reshape(1, D)

    deg_k = _make_deg_kernel()
    agg_k = _make_agg_kernel()

    x_p = jnp.pad(x, ((0, NP - N_NODES), (0, 0)))
    degp = deg_k(dst_p)
    y1 = _mm1(degp, x_p, W1)
    agg1 = agg_k(y1, src_p, dst_p)
    y2 = _mm2(degp, agg1, y1, W2, b1r)
    agg2 = agg_k(y2, src_p, dst_p)
    return _mm3(degp, agg2, y2, b2r)[:N_NODES]
